# baseline (speedup 1.0000x reference)
import functools

import jax
import jax.numpy as jnp
from jax.experimental import pallas as pl
from jax.experimental.pallas import tpu as pltpu

N_ROWS = 8192
IN_DIM = 769
H = 64
A = 32
S = 64
OUT_W = A * S

TILE = 256
GRID = N_ROWS // TILE


def _fused_body(x_ref, w1_ref, b1_ref, w2_ref, b2_ref, w3_ref, m4_ref, o_ref):
    xt = x_ref[...]
    g = xt[:, IN_DIM - 1:IN_DIM]
    ml = ((g >= 0.0) & (g < 0.5)).astype(jnp.float32)
    mm = ((g >= 0.5) & (g < 0.75)).astype(jnp.float32)
    mh = ((g >= 0.75) & (g <= 1.0)).astype(jnp.float32)

    h1 = jnp.dot(xt.astype(jnp.bfloat16), w1_ref[...],
                 preferred_element_type=jnp.float32) + b1_ref[...]
    h1 = jnp.maximum(h1, 0.0)
    h1m = jnp.concatenate(
        [h1[:, 0:H] * ml, h1[:, H:2 * H] * mm, h1[:, 2 * H:3 * H] * mh], axis=1)

    h2 = jnp.dot(h1m.astype(jnp.bfloat16), w2_ref[...],
                 preferred_element_type=jnp.float32) + b2_ref[...]
    h2 = jnp.maximum(h2, 0.0)

    zeros_pad = jnp.zeros((TILE, 256 - 3 * H - 3), jnp.float32)
    aug = jnp.concatenate(
        [h2[:, 0:H] * ml, h2[:, H:2 * H] * mm, h2[:, 2 * H:3 * H] * mh,
         ml, mm, mh, zeros_pad], axis=1)

    y = jnp.dot(aug.astype(jnp.bfloat16), w3_ref[...],
                preferred_element_type=jnp.float32)
    y = jnp.maximum(y, 0.0)

    y8 = y.reshape(TILE * 8, 256)
    o = jnp.dot(y8.astype(jnp.bfloat16), m4_ref[...],
                preferred_element_type=jnp.float32)
    o4 = jnp.stack([o[:, 0:S], o[:, S:2 * S], o[:, 2 * S:3 * S],
                    o[:, 3 * S:4 * S]], axis=1)
    o_ref[...] = o4.reshape(TILE, A, S)


@functools.partial(jax.jit, static_argnames=())
def _prep_and_run(x, lW1, lb1, lW2, lb2, lW3, lb3,
                  mW1, mb1, mW2, mb2, mW3, mb3,
                  hW1, hb1, hW2, hb2, hW3, hb3):
    f32 = jnp.float32
    bf16 = jnp.bfloat16

    w1c = jnp.concatenate([lW1, mW1, hW1], axis=1).astype(bf16)
    b1c = jnp.concatenate([lb1, mb1, hb1]).reshape(1, 3 * H).astype(f32)

    zb = jnp.zeros((H, H), f32)
    w2bd = jnp.block([[lW2, zb, zb], [zb, mW2, zb], [zb, zb, hW2]]).astype(bf16)
    b2c = jnp.concatenate([lb2, mb2, hb2]).reshape(1, 3 * H).astype(f32)

    w3v = jnp.concatenate(
        [lW3, mW3, hW3, lb3.reshape(1, OUT_W), mb3.reshape(1, OUT_W),
         hb3.reshape(1, OUT_W), jnp.zeros((256 - 3 * H - 3, OUT_W), f32)],
        axis=0).astype(bf16)

    jj = jax.lax.broadcasted_iota(jnp.int32, (256, 256), 0)
    kk = jax.lax.broadcasted_iota(jnp.int32, (256, 256), 1)
    m4 = (((jj // S) == (kk // S)) & ((jj % S) + (kk % S) <= S - 1)).astype(bf16)

    out = pl.pallas_call(
        _fused_body,
        grid=(GRID,),
        in_specs=[
            pl.BlockSpec((TILE, IN_DIM), lambda t: (t, 0)),
            pl.BlockSpec((IN_DIM, 3 * H), lambda t: (0, 0)),
            pl.BlockSpec((1, 3 * H), lambda t: (0, 0)),
            pl.BlockSpec((3 * H, 3 * H), lambda t: (0, 0)),
            pl.BlockSpec((1, 3 * H), lambda t: (0, 0)),
            pl.BlockSpec((256, OUT_W), lambda t: (0, 0)),
            pl.BlockSpec((256, 256), lambda t: (0, 0)),
        ],
        out_specs=pl.BlockSpec((TILE, A, S), lambda t: (t, 0, 0)),
        out_shape=jax.ShapeDtypeStruct((N_ROWS, A, S), f32),
    )(x, w1c, b1c, w2bd, b2c, w3v, m4)
    return out


def kernel(x, lW1, lb1, lW2, lb2, lW3, lb3, mW1, mb1, mW2, mb2, mW3, mb3,
           hW1, hb1, hW2, hb2, hW3, hb3):
    return _prep_and_run(x, lW1, lb1, lW2, lb2, lW3, lb3,
                         mW1, mb1, mW2, mb2, mW3, mb3,
                         hW1, hb1, hW2, hb2, hW3, hb3)


# speedup vs baseline: 1.5811x; 1.5811x over previous
import functools

import jax
import jax.numpy as jnp
from jax.experimental import pallas as pl
from jax.experimental.pallas import tpu as pltpu

N_ROWS = 8192
IN_DIM = 769
H = 64
A = 32
S = 64
OUT_W = A * S

TILE = 256
GRID = N_ROWS // TILE


def _fused_body(x_ref, w1_ref, b1_ref, w2_ref, b2_ref, w3_ref, m4_ref, o_ref):
    xt = x_ref[...]
    g = xt[:, IN_DIM - 1:IN_DIM]
    ml = ((g >= 0.0) & (g < 0.5)).astype(jnp.float32)
    mm = ((g >= 0.5) & (g < 0.75)).astype(jnp.float32)
    mh = ((g >= 0.75) & (g <= 1.0)).astype(jnp.float32)

    h1 = jnp.dot(xt.astype(jnp.bfloat16), w1_ref[...],
                 preferred_element_type=jnp.float32) + b1_ref[...]
    h1 = jnp.maximum(h1, 0.0)
    h1m = jnp.concatenate(
        [h1[:, 0:H] * ml, h1[:, H:2 * H] * mm, h1[:, 2 * H:3 * H] * mh], axis=1)

    h2 = jnp.dot(h1m.astype(jnp.bfloat16), w2_ref[...],
                 preferred_element_type=jnp.float32) + b2_ref[...]
    h2 = jnp.maximum(h2, 0.0)

    zeros_pad = jnp.zeros((TILE, 256 - 3 * H - 3), jnp.float32)
    aug = jnp.concatenate(
        [h2[:, 0:H] * ml, h2[:, H:2 * H] * mm, h2[:, 2 * H:3 * H] * mh,
         ml, mm, mh, zeros_pad], axis=1)

    y = jnp.dot(aug.astype(jnp.bfloat16), w3_ref[...],
                preferred_element_type=jnp.float32)
    y = jnp.maximum(y, 0.0)

    y8 = y.reshape(TILE * 8, 256)
    o = jnp.dot(y8.astype(jnp.bfloat16), m4_ref[...],
                preferred_element_type=jnp.float32)
    o_ref[...] = jnp.concatenate(
        [o[:, S * b:S * (b + 1)].reshape(TILE, 8, S) for b in range(4)], axis=1)


@functools.partial(jax.jit, static_argnames=())
def _prep_and_run(x, lW1, lb1, lW2, lb2, lW3, lb3,
                  mW1, mb1, mW2, mb2, mW3, mb3,
                  hW1, hb1, hW2, hb2, hW3, hb3):
    f32 = jnp.float32
    bf16 = jnp.bfloat16

    w1c = jnp.concatenate([lW1, mW1, hW1], axis=1).astype(bf16)
    b1c = jnp.concatenate([lb1, mb1, hb1]).reshape(1, 3 * H).astype(f32)

    zb = jnp.zeros((H, H), f32)
    w2bd = jnp.block([[lW2, zb, zb], [zb, mW2, zb], [zb, zb, hW2]]).astype(bf16)
    b2c = jnp.concatenate([lb2, mb2, hb2]).reshape(1, 3 * H).astype(f32)

    w3v = jnp.concatenate(
        [lW3, mW3, hW3, lb3.reshape(1, OUT_W), mb3.reshape(1, OUT_W),
         hb3.reshape(1, OUT_W), jnp.zeros((256 - 3 * H - 3, OUT_W), f32)],
        axis=0).astype(bf16)
    w3v = w3v.reshape(256, 4, 8, S).transpose(0, 2, 1, 3).reshape(256, OUT_W)

    jj = jax.lax.broadcasted_iota(jnp.int32, (256, 256), 0)
    kk = jax.lax.broadcasted_iota(jnp.int32, (256, 256), 1)
    m4 = (((jj // S) == (kk // S)) & ((jj % S) + (kk % S) <= S - 1)).astype(bf16)

    out = pl.pallas_call(
        _fused_body,
        grid=(GRID,),
        in_specs=[
            pl.BlockSpec((TILE, IN_DIM), lambda t: (t, 0)),
            pl.BlockSpec((IN_DIM, 3 * H), lambda t: (0, 0)),
            pl.BlockSpec((1, 3 * H), lambda t: (0, 0)),
            pl.BlockSpec((3 * H, 3 * H), lambda t: (0, 0)),
            pl.BlockSpec((1, 3 * H), lambda t: (0, 0)),
            pl.BlockSpec((256, OUT_W), lambda t: (0, 0)),
            pl.BlockSpec((256, 256), lambda t: (0, 0)),
        ],
        out_specs=pl.BlockSpec((TILE, A, S), lambda t: (t, 0, 0)),
        out_shape=jax.ShapeDtypeStruct((N_ROWS, A, S), f32),
    )(x, w1c, b1c, w2bd, b2c, w3v, m4)
    return out


def kernel(x, lW1, lb1, lW2, lb2, lW3, lb3, mW1, mb1, mW2, mb2, mW3, mb3,
           hW1, hb1, hW2, hb2, hW3, hb3):
    return _prep_and_run(x, lW1, lb1, lW2, lb2, lW3, lb3,
                         mW1, mb1, mW2, mb2, mW3, mb3,
                         hW1, hb1, hW2, hb2, hW3, hb3)


# speedup vs baseline: 1.8989x; 1.2010x over previous
import functools

import jax
import jax.numpy as jnp
from jax.experimental import pallas as pl
from jax.experimental.pallas import tpu as pltpu

N_ROWS = 8192
IN_DIM = 769
H = 64
A = 32
S = 64
OUT_W = A * S

TILE = 256
GRID = N_ROWS // TILE


def _fused_body(x_ref, w1_ref, b1_ref, w2_ref, b2_ref, w3_ref, m4_ref, o_ref):
    xt = x_ref[...]
    g = xt[:, IN_DIM - 1:IN_DIM]
    ml = ((g >= 0.0) & (g < 0.5)).astype(jnp.float32)
    mm = ((g >= 0.5) & (g < 0.75)).astype(jnp.float32)
    mh = ((g >= 0.75) & (g <= 1.0)).astype(jnp.float32)

    h1 = jnp.dot(xt.astype(jnp.bfloat16), w1_ref[...],
                 preferred_element_type=jnp.float32) + b1_ref[...]
    h1 = jnp.maximum(h1, 0.0)
    h1m = jnp.concatenate(
        [h1[:, 0:H] * ml, h1[:, H:2 * H] * mm, h1[:, 2 * H:3 * H] * mh], axis=1)

    h2 = jnp.dot(h1m.astype(jnp.bfloat16), w2_ref[...],
                 preferred_element_type=jnp.float32) + b2_ref[...]
    h2 = jnp.maximum(h2, 0.0)

    zeros_pad = jnp.zeros((TILE, 256 - 3 * H - 3), jnp.float32)
    aug = jnp.concatenate(
        [h2[:, 0:H] * ml, h2[:, H:2 * H] * mm, h2[:, 2 * H:3 * H] * mh,
         ml, mm, mh, zeros_pad], axis=1)

    y = jnp.dot(aug.astype(jnp.bfloat16), w3_ref[...],
                preferred_element_type=jnp.float32)
    y = jnp.maximum(y, 0.0)

    y8 = y.reshape(TILE * 8, 256)
    o = jnp.dot(y8.astype(jnp.bfloat16), m4_ref[...],
                preferred_element_type=jnp.float32)
    o_ref[...] = o.reshape(TILE, OUT_W)


@functools.partial(jax.jit, static_argnames=())
def _prep_and_run(x, lW1, lb1, lW2, lb2, lW3, lb3,
                  mW1, mb1, mW2, mb2, mW3, mb3,
                  hW1, hb1, hW2, hb2, hW3, hb3):
    f32 = jnp.float32
    bf16 = jnp.bfloat16

    w1c = jnp.concatenate([lW1, mW1, hW1], axis=1).astype(bf16)
    b1c = jnp.concatenate([lb1, mb1, hb1]).reshape(1, 3 * H).astype(f32)

    zb = jnp.zeros((H, H), f32)
    w2bd = jnp.block([[lW2, zb, zb], [zb, mW2, zb], [zb, zb, hW2]]).astype(bf16)
    b2c = jnp.concatenate([lb2, mb2, hb2]).reshape(1, 3 * H).astype(f32)

    w3v = jnp.concatenate(
        [lW3, mW3, hW3, lb3.reshape(1, OUT_W), mb3.reshape(1, OUT_W),
         hb3.reshape(1, OUT_W), jnp.zeros((256 - 3 * H - 3, OUT_W), f32)],
        axis=0).astype(bf16)

    jj = jax.lax.broadcasted_iota(jnp.int32, (256, 256), 0)
    kk = jax.lax.broadcasted_iota(jnp.int32, (256, 256), 1)
    m4 = (((jj // S) == (kk // S)) & ((jj % S) + (kk % S) <= S - 1)).astype(bf16)

    out = pl.pallas_call(
        _fused_body,
        grid=(GRID,),
        in_specs=[
            pl.BlockSpec((TILE, IN_DIM), lambda t: (t, 0)),
            pl.BlockSpec((IN_DIM, 3 * H), lambda t: (0, 0)),
            pl.BlockSpec((1, 3 * H), lambda t: (0, 0)),
            pl.BlockSpec((3 * H, 3 * H), lambda t: (0, 0)),
            pl.BlockSpec((1, 3 * H), lambda t: (0, 0)),
            pl.BlockSpec((256, OUT_W), lambda t: (0, 0)),
            pl.BlockSpec((256, 256), lambda t: (0, 0)),
        ],
        out_specs=pl.BlockSpec((TILE, OUT_W), lambda t: (t, 0)),
        out_shape=jax.ShapeDtypeStruct((N_ROWS, OUT_W), f32),
    )(x, w1c, b1c, w2bd, b2c, w3v, m4)
    return out.reshape(N_ROWS, A, S)


def kernel(x, lW1, lb1, lW2, lb2, lW3, lb3, mW1, mb1, mW2, mb2, mW3, mb3,
           hW1, hb1, hW2, hb2, hW3, hb3):
    return _prep_and_run(x, lW1, lb1, lW2, lb2, lW3, lb3,
                         mW1, mb1, mW2, mb2, mW3, mb3,
                         hW1, hb1, hW2, hb2, hW3, hb3)


# speedup vs baseline: 4.2219x; 2.2233x over previous
import functools

import jax
import jax.numpy as jnp
from jax.experimental import pallas as pl

N_ROWS = 8192
IN_DIM = 769
H = 64
A = 32
S = 64
OUT_W = A * S

TILE = 256
GRID = N_ROWS // TILE


def _fused_body(x_ref, w1_ref, b1_ref, w2_ref, b2_ref, w3_ref, m4_ref, o_ref):
    xt = x_ref[...]
    g = xt[IN_DIM - 1:IN_DIM, :]
    ml = ((g >= 0.0) & (g < 0.5)).astype(jnp.float32)
    mm = ((g >= 0.5) & (g < 0.75)).astype(jnp.float32)
    mh = ((g >= 0.75) & (g <= 1.0)).astype(jnp.float32)

    h1 = jnp.dot(w1_ref[...], xt.astype(jnp.bfloat16),
                 preferred_element_type=jnp.float32) + b1_ref[...]
    h1 = jnp.maximum(h1, 0.0)
    h1m = jnp.concatenate(
        [h1[0:H] * ml, h1[H:2 * H] * mm, h1[2 * H:3 * H] * mh], axis=0)

    h2 = jnp.dot(w2_ref[...], h1m.astype(jnp.bfloat16),
                 preferred_element_type=jnp.float32) + b2_ref[...]
    h2 = jnp.maximum(h2, 0.0)

    zeros_pad = jnp.zeros((256 - 3 * H - 3, TILE), jnp.float32)
    aug = jnp.concatenate(
        [h2[0:H] * ml, h2[H:2 * H] * mm, h2[2 * H:3 * H] * mh,
         ml, mm, mh, zeros_pad], axis=0)

    y = jnp.dot(w3_ref[...], aug.astype(jnp.bfloat16),
                preferred_element_type=jnp.float32)
    y = jnp.maximum(y, 0.0)

    yb = y.astype(jnp.bfloat16)
    m4 = m4_ref[...]
    for m in range(8):
        o_ref[m * 256:(m + 1) * 256, :] = jnp.dot(
            m4, yb[m * 256:(m + 1) * 256, :],
            preferred_element_type=jnp.float32)


@functools.partial(jax.jit, static_argnames=())
def _prep_and_run(x, lW1, lb1, lW2, lb2, lW3, lb3,
                  mW1, mb1, mW2, mb2, mW3, mb3,
                  hW1, hb1, hW2, hb2, hW3, hb3):
    f32 = jnp.float32
    bf16 = jnp.bfloat16

    xT = x.T

    w1t = jnp.concatenate([lW1.T, mW1.T, hW1.T], axis=0).astype(bf16)
    b1t = jnp.concatenate([lb1, mb1, hb1]).reshape(3 * H, 1).astype(f32)

    zb = jnp.zeros((H, H), f32)
    w2t = jnp.block([[lW2.T, zb, zb], [zb, mW2.T, zb],
                     [zb, zb, hW2.T]]).astype(bf16)
    b2t = jnp.concatenate([lb2, mb2, hb2]).reshape(3 * H, 1).astype(f32)

    w3t = jnp.concatenate(
        [lW3, mW3, hW3, lb3.reshape(1, OUT_W), mb3.reshape(1, OUT_W),
         hb3.reshape(1, OUT_W), jnp.zeros((256 - 3 * H - 3, OUT_W), f32)],
        axis=0).T.astype(bf16)

    jj = jax.lax.broadcasted_iota(jnp.int32, (256, 256), 0)
    kk = jax.lax.broadcasted_iota(jnp.int32, (256, 256), 1)
    m4 = (((jj // S) == (kk // S)) & ((jj % S) + (kk % S) <= S - 1)).astype(bf16)

    out = pl.pallas_call(
        _fused_body,
        grid=(GRID,),
        in_specs=[
            pl.BlockSpec((IN_DIM, TILE), lambda t: (0, t)),
            pl.BlockSpec((3 * H, IN_DIM), lambda t: (0, 0)),
            pl.BlockSpec((3 * H, 1), lambda t: (0, 0)),
            pl.BlockSpec((3 * H, 3 * H), lambda t: (0, 0)),
            pl.BlockSpec((3 * H, 1), lambda t: (0, 0)),
            pl.BlockSpec((OUT_W, 256), lambda t: (0, 0)),
            pl.BlockSpec((256, 256), lambda t: (0, 0)),
        ],
        out_specs=pl.BlockSpec((OUT_W, TILE), lambda t: (0, t)),
        out_shape=jax.ShapeDtypeStruct((OUT_W, N_ROWS), f32),
    )(xT, w1t, b1t, w2t, b2t, w3t, m4)
    return out.reshape(A, S, N_ROWS).transpose(2, 0, 1)


def kernel(x, lW1, lb1, lW2, lb2, lW3, lb3, mW1, mb1, mW2, mb2, mW3, mb3,
           hW1, hb1, hW2, hb2, hW3, hb3):
    return _prep_and_run(x, lW1, lb1, lW2, lb2, lW3, lb3,
                         mW1, mb1, mW2, mb2, mW3, mb3,
                         hW1, hb1, hW2, hb2, hW3, hb3)


# speedup vs baseline: 5.2204x; 1.2365x over previous
import functools

import jax
import jax.numpy as jnp
from jax.experimental import pallas as pl

N_ROWS = 8192
IN_DIM = 769
H = 64
A = 32
S = 64
OUT_W = A * S

TILE = 512
GRID = N_ROWS // TILE


def _fused_body(x_ref, w1_ref, b1_ref, w2_ref, b2_ref, w3_ref, m4_ref, o_ref):
    xt = x_ref[...]
    g = xt[IN_DIM - 1:IN_DIM, :]
    ml = ((g >= 0.0) & (g < 0.5)).astype(jnp.float32)
    mm = ((g >= 0.5) & (g < 0.75)).astype(jnp.float32)
    mh = ((g >= 0.75) & (g <= 1.0)).astype(jnp.float32)

    h1 = jnp.dot(w1_ref[...], xt.astype(jnp.bfloat16),
                 preferred_element_type=jnp.float32) + b1_ref[...]
    h1 = jnp.maximum(h1, 0.0)
    h1m = jnp.concatenate(
        [h1[0:H] * ml, h1[H:2 * H] * mm, h1[2 * H:3 * H] * mh], axis=0)

    h2 = jnp.dot(w2_ref[...], h1m.astype(jnp.bfloat16),
                 preferred_element_type=jnp.float32) + b2_ref[...]
    h2 = jnp.maximum(h2, 0.0)

    zeros_pad = jnp.zeros((256 - 3 * H - 3, TILE), jnp.float32)
    aug = jnp.concatenate(
        [h2[0:H] * ml, h2[H:2 * H] * mm, h2[2 * H:3 * H] * mh,
         ml, mm, mh, zeros_pad], axis=0)

    y = jnp.dot(w3_ref[...], aug.astype(jnp.bfloat16),
                preferred_element_type=jnp.float32)
    y = jnp.maximum(y, 0.0)

    yb = y.astype(jnp.bfloat16)
    m4 = m4_ref[...]
    for m in range(8):
        o_ref[m * 256:(m + 1) * 256, :] = jnp.dot(
            m4, yb[m * 256:(m + 1) * 256, :],
            preferred_element_type=jnp.float32)


@functools.partial(jax.jit, static_argnames=())
def _prep_and_run(x, lW1, lb1, lW2, lb2, lW3, lb3,
                  mW1, mb1, mW2, mb2, mW3, mb3,
                  hW1, hb1, hW2, hb2, hW3, hb3):
    f32 = jnp.float32
    bf16 = jnp.bfloat16

    xT = x.T

    w1t = jnp.concatenate([lW1.T, mW1.T, hW1.T], axis=0).astype(bf16)
    b1t = jnp.concatenate([lb1, mb1, hb1]).reshape(3 * H, 1).astype(f32)

    zb = jnp.zeros((H, H), f32)
    w2t = jnp.block([[lW2.T, zb, zb], [zb, mW2.T, zb],
                     [zb, zb, hW2.T]]).astype(bf16)
    b2t = jnp.concatenate([lb2, mb2, hb2]).reshape(3 * H, 1).astype(f32)

    w3t = jnp.concatenate(
        [lW3.T, mW3.T, hW3.T, lb3.reshape(OUT_W, 1), mb3.reshape(OUT_W, 1),
         hb3.reshape(OUT_W, 1), jnp.zeros((OUT_W, 256 - 3 * H - 3), f32)],
        axis=1).astype(bf16)

    jj = jax.lax.broadcasted_iota(jnp.int32, (256, 256), 0)
    kk = jax.lax.broadcasted_iota(jnp.int32, (256, 256), 1)
    m4 = (((jj // S) == (kk // S)) & ((jj % S) + (kk % S) <= S - 1)).astype(bf16)

    out = pl.pallas_call(
        _fused_body,
        grid=(GRID,),
        in_specs=[
            pl.BlockSpec((IN_DIM, TILE), lambda t: (0, t)),
            pl.BlockSpec((3 * H, IN_DIM), lambda t: (0, 0)),
            pl.BlockSpec((3 * H, 1), lambda t: (0, 0)),
            pl.BlockSpec((3 * H, 3 * H), lambda t: (0, 0)),
            pl.BlockSpec((3 * H, 1), lambda t: (0, 0)),
            pl.BlockSpec((OUT_W, 256), lambda t: (0, 0)),
            pl.BlockSpec((256, 256), lambda t: (0, 0)),
        ],
        out_specs=pl.BlockSpec((OUT_W, TILE), lambda t: (0, t)),
        out_shape=jax.ShapeDtypeStruct((OUT_W, N_ROWS), f32),
    )(xT, w1t, b1t, w2t, b2t, w3t, m4)
    return out.reshape(A, S, N_ROWS).transpose(2, 0, 1)


def kernel(x, lW1, lb1, lW2, lb2, lW3, lb3, mW1, mb1, mW2, mb2, mW3, mb3,
           hW1, hb1, hW2, hb2, hW3, hb3):
    return _prep_and_run(x, lW1, lb1, lW2, lb2, lW3, lb3,
                         mW1, mb1, mW2, mb2, mW3, mb3,
                         hW1, hb1, hW2, hb2, hW3, hb3)


# speedup vs baseline: 5.7599x; 1.1034x over previous
import functools

import jax
import jax.numpy as jnp
from jax.experimental import pallas as pl

N_ROWS = 8192
IN_DIM = 769
H = 64
A = 32
S = 64
OUT_W = A * S

TILE = 1024
GRID = N_ROWS // TILE


def _fused_body(x_ref, w1_ref, b1_ref, w2_ref, b2_ref, w3_ref, m4_ref, o_ref):
    xt = x_ref[...]
    g = xt[IN_DIM - 1:IN_DIM, :]
    ml = ((g >= 0.0) & (g < 0.5)).astype(jnp.float32)
    mm = ((g >= 0.5) & (g < 0.75)).astype(jnp.float32)
    mh = ((g >= 0.75) & (g <= 1.0)).astype(jnp.float32)

    h1 = jnp.dot(w1_ref[...], xt.astype(jnp.bfloat16),
                 preferred_element_type=jnp.float32) + b1_ref[...]
    h1 = jnp.maximum(h1, 0.0)
    h1m = jnp.concatenate(
        [h1[0:H] * ml, h1[H:2 * H] * mm, h1[2 * H:3 * H] * mh], axis=0)

    h2 = jnp.dot(w2_ref[...], h1m.astype(jnp.bfloat16),
                 preferred_element_type=jnp.float32) + b2_ref[...]
    h2 = jnp.maximum(h2, 0.0)

    zeros_pad = jnp.zeros((256 - 3 * H - 3, TILE), jnp.float32)
    aug = jnp.concatenate(
        [h2[0:H] * ml, h2[H:2 * H] * mm, h2[2 * H:3 * H] * mh,
         ml, mm, mh, zeros_pad], axis=0)

    y = jnp.dot(w3_ref[...], aug.astype(jnp.bfloat16),
                preferred_element_type=jnp.float32)
    y = jnp.maximum(y, 0.0)

    yb = y.astype(jnp.bfloat16)
    m4 = m4_ref[...]
    for m in range(8):
        o_ref[m * 256:(m + 1) * 256, :] = jnp.dot(
            m4, yb[m * 256:(m + 1) * 256, :],
            preferred_element_type=jnp.float32)


@functools.partial(jax.jit, static_argnames=())
def _prep_and_run(x, lW1, lb1, lW2, lb2, lW3, lb3,
                  mW1, mb1, mW2, mb2, mW3, mb3,
                  hW1, hb1, hW2, hb2, hW3, hb3):
    f32 = jnp.float32
    bf16 = jnp.bfloat16

    xT = x.T

    w1t = jnp.concatenate([lW1.T, mW1.T, hW1.T], axis=0).astype(bf16)
    b1t = jnp.concatenate([lb1, mb1, hb1]).reshape(3 * H, 1).astype(f32)

    zb = jnp.zeros((H, H), f32)
    w2t = jnp.block([[lW2.T, zb, zb], [zb, mW2.T, zb],
                     [zb, zb, hW2.T]]).astype(bf16)
    b2t = jnp.concatenate([lb2, mb2, hb2]).reshape(3 * H, 1).astype(f32)

    w3t = jnp.concatenate(
        [lW3.T, mW3.T, hW3.T, lb3.reshape(OUT_W, 1), mb3.reshape(OUT_W, 1),
         hb3.reshape(OUT_W, 1), jnp.zeros((OUT_W, 256 - 3 * H - 3), f32)],
        axis=1).astype(bf16)

    jj = jax.lax.broadcasted_iota(jnp.int32, (256, 256), 0)
    kk = jax.lax.broadcasted_iota(jnp.int32, (256, 256), 1)
    m4 = (((jj // S) == (kk // S)) & ((jj % S) + (kk % S) <= S - 1)).astype(bf16)

    out = pl.pallas_call(
        _fused_body,
        grid=(GRID,),
        in_specs=[
            pl.BlockSpec((IN_DIM, TILE), lambda t: (0, t)),
            pl.BlockSpec((3 * H, IN_DIM), lambda t: (0, 0)),
            pl.BlockSpec((3 * H, 1), lambda t: (0, 0)),
            pl.BlockSpec((3 * H, 3 * H), lambda t: (0, 0)),
            pl.BlockSpec((3 * H, 1), lambda t: (0, 0)),
            pl.BlockSpec((OUT_W, 256), lambda t: (0, 0)),
            pl.BlockSpec((256, 256), lambda t: (0, 0)),
        ],
        out_specs=pl.BlockSpec((OUT_W, TILE), lambda t: (0, t)),
        out_shape=jax.ShapeDtypeStruct((OUT_W, N_ROWS), f32),
    )(xT, w1t, b1t, w2t, b2t, w3t, m4)
    return out.reshape(A, S, N_ROWS).transpose(2, 0, 1)


def kernel(x, lW1, lb1, lW2, lb2, lW3, lb3, mW1, mb1, mW2, mb2, mW3, mb3,
           hW1, hb1, hW2, hb2, hW3, hb3):
    return _prep_and_run(x, lW1, lb1, lW2, lb2, lW3, lb3,
                         mW1, mb1, mW2, mb2, mW3, mb3,
                         hW1, hb1, hW2, hb2, hW3, hb3)
